# baseline (speedup 1.0000x reference)
import functools

import jax
import jax.numpy as jnp
from jax import lax
from jax.experimental import pallas as pl
from jax.experimental.pallas import tpu as pltpu
from jax.experimental.pallas import tpu_sc as plsc

N_ENTITY = 1000000
N_RELATION = 1000
HIDDEN = 64
BATCH = 4096

_NC, _NS = 2, 16
_NW = _NC * _NS
_BW = BATCH // _NW
_NTC = (N_ENTITY + 127) // 128
_KCAP = 3 * BATCH
_RING = 64

_mesh = plsc.VectorSubcoreMesh(core_axis_name="c", subcore_axis_name="s")


@functools.partial(
    pl.kernel,
    mesh=_mesh,
    compiler_params=pltpu.CompilerParams(use_tc_tiling_on_sc=True,
                                         needs_layout_passes=False),
    out_type=(
        jax.ShapeDtypeStruct((BATCH * HIDDEN,), jnp.float32),
        jax.ShapeDtypeStruct((BATCH * HIDDEN,), jnp.float32),
        jax.ShapeDtypeStruct((BATCH, HIDDEN), jnp.float32),
        jax.ShapeDtypeStruct((BATCH * HIDDEN,), jnp.float32),
        jax.ShapeDtypeStruct((BATCH,), jnp.float32),
        jax.ShapeDtypeStruct((BATCH, 2 * HIDDEN), jnp.float32),
        jax.ShapeDtypeStruct((BATCH, HIDDEN), jnp.float32),
        jax.ShapeDtypeStruct((BATCH,), jnp.float32),
        jax.ShapeDtypeStruct((BATCH,), jnp.float32),
        jax.ShapeDtypeStruct((BATCH,), jnp.float32),
    ),
    scratch_types=(
        pltpu.VMEM((_BW,), jnp.int32),
        pltpu.VMEM((_BW,), jnp.int32),
        pltpu.VMEM((_BW,), jnp.int32),
        pltpu.VMEM((_BW,), jnp.int32),
        pltpu.VMEM((_BW,), jnp.int32),
        pltpu.VMEM((BATCH,), jnp.int32),
        pltpu.VMEM((BATCH,), jnp.int32),
        pltpu.VMEM((BATCH,), jnp.int32),
        pltpu.VMEM((_KCAP,), jnp.int32),
        pltpu.VMEM((_KCAP,), jnp.int32),
        pltpu.SMEM((256,), jnp.int32),
        pltpu.SMEM((256,), jnp.int32),
        pltpu.SMEM((256,), jnp.int32),
        pltpu.VMEM((64, 256), jnp.float32),
        pltpu.VMEM((64, 256), jnp.float32),
        pltpu.VMEM((_RING, HIDDEN), jnp.float32),
        pltpu.VMEM((_BW, 128), jnp.float32),
        pltpu.VMEM((_BW, HIDDEN), jnp.float32),
        pltpu.VMEM((_BW, HIDDEN), jnp.float32),
        pltpu.VMEM((_BW,), jnp.float32),
        pltpu.VMEM((_BW,), jnp.float32),
        pltpu.VMEM((_BW,), jnp.float32),
        pltpu.VMEM((_BW,), jnp.float32),
        pltpu.SemaphoreType.DMA,
        pltpu.SemaphoreType.DMA,
        pltpu.SemaphoreType.DMA,
        pltpu.SemaphoreType.DMA,
        pltpu.SemaphoreType.DMA,
        pltpu.SemaphoreType.DMA,
    ),
)
def _gather_all(etT, rel2, diag, curv1, ctx2, hb1, tb1, head, tail, rel, neg,
                head_o, tail_o, rel_o, neg_o, curv_o, diag_o, ctx_o,
                hb_o, tb_o, ntb_o,
                ridx, rp, hidx, tidx, nidx, hfull, tfull, nfull,
                hits, hits2, colsm, colstart, coloff, chunk0, chunk1, ring,
                lineA, rrow, xrow, crow, hbrow, tbrow, ntbrow,
                gsem, semA, csem0, csem1, rowsem, ssem):
    wid = lax.axis_index("s") * _NC + lax.axis_index("c")
    base = wid * _BW
    sl = pl.ds(base, _BW)
    iota16 = lax.iota(jnp.int32, 16)

    pltpu.sync_copy(rel.at[sl], ridx)
    pltpu.sync_copy(head.at[sl], hidx)
    pltpu.sync_copy(tail.at[sl], tidx)
    pltpu.sync_copy(neg.at[sl], nidx)

    def mkpairs(i, _):
        s16 = pl.ds(i * 16, 16)
        rp[s16] = ridx[s16] >> 1
        return 0
    lax.fori_loop(0, _BW // 16, mkpairs, 0, unroll=True)

    small = [
        pltpu.async_copy(curv1.at[ridx], crow, gsem),
        pltpu.async_copy(hb1.at[hidx], hbrow, gsem),
        pltpu.async_copy(tb1.at[tidx], tbrow, gsem),
        pltpu.async_copy(tb1.at[nidx], ntbrow, gsem),
    ]
    dA = pltpu.async_copy(rel2.at[rp], lineA, semA)

    def sel_table(line, idxv, row):
        for k in range(_BW // 16):
            hvec = iota16 + (16 * k)
            colb = (idxv[pl.ds(16 * k, 16)] & 1) * HIDDEN

            def cbody(c, _):
                v = plsc.load_gather(line, [hvec, colb + c])
                cvec = jnp.full((16,), c, dtype=jnp.int32)
                plsc.store_scatter(row, [hvec, cvec], v)
                return 0
            lax.fori_loop(0, HIDDEN, cbody, 0)

    pltpu.sync_copy(head, hfull)
    pltpu.sync_copy(tail, tfull)
    pltpu.sync_copy(neg, nfull)

    t0 = (wid * _NTC) // _NW
    t1 = ((wid + 1) * _NTC) // _NW

    def build(tblref, tblid, kcnt0):
        def chunk(j, kcnt):
            r = tblref[pl.ds(j * 16, 16)]
            tc = r >> 7
            m = (tc >= t0) & (tc < t1)
            n = lax.reduce_max(plsc.all_reduce_population_count(m), (0,))

            def have():
                pos = kcnt + plsc.cumsum(m.astype(jnp.int32)) - 1
                ea = ((tc - t0) << 7) | (r & 127)
                eb = (16 * j + iota16) | (tblid << 12)
                plsc.store_scatter(hits, [pos], ea | (eb << 15), mask=m)
            pl.when(n > 0)(have)
            return kcnt + n
        return lax.fori_loop(0, BATCH // 16, chunk, kcnt0)

    kcnt = build(hfull, 0, jnp.int32(0))
    kcnt = build(tfull, 1, kcnt)
    kcnt = build(nfull, 2, kcnt)
    kchunks = (kcnt + 15) >> 4

    ncols = t1 - t0
    dummy = hb1.at[pl.ds(0, HIDDEN)]
    zeros16 = jnp.zeros((16,), jnp.int32)

    def zerocol(i, _):
        colsm[i] = 0
        return 0
    lax.fori_loop(0, 256, zerocol, 0)

    def count_chunk(j, _):
        a = hits[pl.ds(j * 16, 16)]
        valid = ((j * 16 + iota16) < kcnt).astype(jnp.int32)
        trel = (a & 0x7FFF) >> 7
        for i in range(16):
            colsm[trel[i]] = colsm[trel[i]] + valid[i]
        return 0
    lax.fori_loop(0, kchunks, count_chunk, 0)

    def prefix(i, run):
        colstart[i] = run
        coloff[i] = run
        return run + colsm[i]
    lax.fori_loop(0, 256, prefix, jnp.int32(0))

    def place_chunk(j, _):
        a = hits[pl.ds(j * 16, 16)]
        validb = (j * 16 + iota16) < kcnt
        valid = validb.astype(jnp.int32)
        trel = (a & 0x7FFF) >> 7
        pos = zeros16
        for i in range(16):
            p = coloff[trel[i]]
            coloff[trel[i]] = p + valid[i]
            pos = jnp.where(iota16 == i, p, pos)
        plsc.store_scatter(hits2, [pos], a, mask=validb)
        return 0
    lax.fori_loop(0, kchunks, place_chunk, 0)

    dA.wait()
    sel_table(lineA, ridx, rrow)
    dX = pltpu.async_copy(ctx2.at[rp], lineA, semA)
    st_rel = pltpu.async_copy(rrow, rel_o.at[sl], ssem)
    dX.wait()
    sel_table(lineA, ridx, xrow)
    dD = pltpu.async_copy(diag.at[ridx], lineA, semA)
    st_ctx = pltpu.async_copy(xrow, ctx_o.at[sl], ssem)

    def process_col(seg_base, seg_cnt, chunk, loff, state):
        def kchunk(j, st):
            a = hits2[pl.ds(seg_base + j * 16, 16)]
            m = (j * 16 + iota16) < seg_cnt
            n = jnp.minimum(seg_cnt - j * 16, 16)

            def have(st2):
                iss2, drn2 = st2
                cur = iss2
                for i in range(16):
                    mi = (j * 16 + i) < seg_cnt
                    e = a[i]
                    b = (e >> 15) & 4095
                    tbl = e >> 27
                    slot = cur & (_RING - 1)

                    @pl.when(mi)
                    def _():
                        lbc = jnp.full((16,), (e & 127) + loff,
                                       dtype=jnp.int32)
                        for q in range(4):
                            v = plsc.load_gather(chunk, [iota16 + 16 * q, lbc])
                            ring[slot, pl.ds(16 * q, 16)] = v
                    src = ring.at[slot]
                    dst = pl.ds(b * HIDDEN, HIDDEN)

                    @pl.when(mi & (tbl == 0))
                    def _():
                        pltpu.async_copy(src, head_o.at[dst], rowsem)

                    @pl.when(mi & (tbl == 1))
                    def _():
                        pltpu.async_copy(src, tail_o.at[dst], rowsem)

                    @pl.when(mi & (tbl == 2))
                    def _():
                        pltpu.async_copy(src, neg_o.at[dst], rowsem)
                    cur = cur + mi.astype(jnp.int32)
                iss2 = cur

                def drain_some(st4):
                    iss4, drn4 = st4

                    def dr(i2, d):
                        pltpu.make_async_copy(dummy, ring.at[0], rowsem).wait()
                        return d + 1
                    drn4 = lax.fori_loop(0, 16, dr, drn4)
                    return (iss4, drn4)
                return lax.cond(iss2 - drn2 >= _RING - 32, drain_some,
                                lambda s: s, (iss2, drn2))
            return lax.cond(n > 0, have, lambda s: s, st)
        return lax.fori_loop(0, (seg_cnt + 15) >> 4, kchunk, state)

    def ubase(c):
        return jnp.minimum(t0 + c, _NTC - 2)

    def unit(c0, chunk, st):
        sh = ((t0 + c0) - ubase(c0)) * 128
        st = process_col(colstart[c0], colsm[c0], chunk, sh, st)

        def second(st2):
            return process_col(colstart[c0 + 1], colsm[c0 + 1], chunk,
                               sh + 128, st2)
        return lax.cond(c0 + 1 < ncols, second, lambda s2: s2, st)

    @pl.when(0 < ncols)
    def _():
        pltpu.async_copy(etT.at[:, pl.ds(ubase(0) * 128, 256)], chunk0, csem0)

    def pairbody(i, st):
        uA = 4 * i
        uB = uA + 2

        @pl.when(uB < ncols)
        def _():
            pltpu.async_copy(etT.at[:, pl.ds(ubase(uB) * 128, 256)],
                             chunk1, csem1)

        def doA(st2):
            pltpu.make_async_copy(etT.at[:, pl.ds(0, 256)], chunk0,
                                  csem0).wait()
            st2 = unit(uA, chunk0, st2)

            @pl.when(uB + 2 < ncols)
            def _():
                pltpu.async_copy(etT.at[:, pl.ds(ubase(uB + 2) * 128, 256)],
                                 chunk0, csem0)
            return st2
        st = lax.cond(uA < ncols, doA, lambda s2: s2, st)

        def doB(st2):
            pltpu.make_async_copy(etT.at[:, pl.ds(0, 256)], chunk1,
                                  csem1).wait()
            return unit(uB, chunk1, st2)
        return lax.cond(uB < ncols, doB, lambda s2: s2, st)

    issued, drained = lax.fori_loop(0, (_NTC // _NW) // 4 + 2, pairbody,
                                    (jnp.int32(0), jnp.int32(0)))

    def drfin(i, d):
        pltpu.make_async_copy(dummy, ring.at[0], rowsem).wait()
        return d + 1
    lax.fori_loop(0, issued - drained, drfin, drained)

    for g in small:
        g.wait()
    dD.wait()
    stores = [
        st_rel, st_ctx,
        pltpu.async_copy(crow, curv_o.at[sl], ssem),
        pltpu.async_copy(lineA, diag_o.at[sl], ssem),
        pltpu.async_copy(hbrow, hb_o.at[sl], ssem),
        pltpu.async_copy(tbrow, tb_o.at[sl], ssem),
        pltpu.async_copy(ntbrow, ntb_o.at[sl], ssem),
    ]
    for s in stores:
        s.wait()


def kernel(entity_emb, relation_emb, relation_diag, curvature, context,
           head_bias, tail_bias, head, tail, rel, neg):
    scale = jnp.array([0.125], dtype=jnp.float32)
    (head_e, tail_e, rel_e, neg_e, curv, rel_diag, ctx,
     h_bias, t_bias, neg_t_bias) = _gather_all(
        entity_emb.T,
        relation_emb.reshape(N_RELATION // 2, 2 * HIDDEN),
        relation_diag,
        curvature.reshape(N_RELATION),
        context.reshape(N_RELATION // 2, 2 * HIDDEN),
        head_bias.reshape(N_ENTITY), tail_bias.reshape(N_ENTITY),
        head.astype(jnp.int32), tail.astype(jnp.int32),
        rel.astype(jnp.int32), neg.astype(jnp.int32))
    return (scale, head_e.reshape(BATCH, HIDDEN), tail_e.reshape(BATCH, HIDDEN),
            rel_e, neg_e.reshape(BATCH, HIDDEN),
            curv.reshape(BATCH, 1), rel_diag, ctx,
            h_bias.reshape(BATCH, 1), t_bias.reshape(BATCH, 1),
            neg_t_bias.reshape(BATCH, 1))


# speedup vs baseline: 1.5195x; 1.5195x over previous
import functools

import jax
import jax.numpy as jnp
from jax import lax
from jax.experimental import pallas as pl
from jax.experimental.pallas import tpu as pltpu
from jax.experimental.pallas import tpu_sc as plsc

N_ENTITY = 1000000
N_RELATION = 1000
HIDDEN = 64
BATCH = 4096

_NC, _NS = 2, 16
_NW = _NC * _NS
_BW = BATCH // _NW
_NTC = (N_ENTITY + 127) // 128
_KCAP = 3 * BATCH
_RING = 128

_mesh = plsc.VectorSubcoreMesh(core_axis_name="c", subcore_axis_name="s")


@functools.partial(
    pl.kernel,
    mesh=_mesh,
    compiler_params=pltpu.CompilerParams(use_tc_tiling_on_sc=True,
                                         needs_layout_passes=False),
    out_type=(
        jax.ShapeDtypeStruct((BATCH * HIDDEN,), jnp.float32),
        jax.ShapeDtypeStruct((BATCH * HIDDEN,), jnp.float32),
        jax.ShapeDtypeStruct((BATCH, HIDDEN), jnp.float32),
        jax.ShapeDtypeStruct((BATCH * HIDDEN,), jnp.float32),
        jax.ShapeDtypeStruct((BATCH,), jnp.float32),
        jax.ShapeDtypeStruct((BATCH, 2 * HIDDEN), jnp.float32),
        jax.ShapeDtypeStruct((BATCH, HIDDEN), jnp.float32),
        jax.ShapeDtypeStruct((BATCH,), jnp.float32),
        jax.ShapeDtypeStruct((BATCH,), jnp.float32),
        jax.ShapeDtypeStruct((BATCH,), jnp.float32),
    ),
    scratch_types=(
        pltpu.VMEM((_BW,), jnp.int32),
        pltpu.VMEM((_BW,), jnp.int32),
        pltpu.VMEM((_BW,), jnp.int32),
        pltpu.VMEM((_BW,), jnp.int32),
        pltpu.VMEM((_BW,), jnp.int32),
        pltpu.VMEM((BATCH,), jnp.int32),
        pltpu.VMEM((BATCH,), jnp.int32),
        pltpu.VMEM((BATCH,), jnp.int32),
        pltpu.VMEM((_KCAP,), jnp.int32),
        pltpu.VMEM((_KCAP,), jnp.int32),
        pltpu.SMEM((256,), jnp.int32),
        pltpu.SMEM((256,), jnp.int32),
        pltpu.SMEM((256,), jnp.int32),
        pltpu.VMEM((64, 128), jnp.float32),
        pltpu.VMEM((64, 128), jnp.float32),
        pltpu.VMEM((_RING, HIDDEN), jnp.float32),
        pltpu.VMEM((_BW, 128), jnp.float32),
        pltpu.VMEM((_BW, HIDDEN), jnp.float32),
        pltpu.VMEM((_BW, HIDDEN), jnp.float32),
        pltpu.VMEM((_BW,), jnp.float32),
        pltpu.VMEM((_BW,), jnp.float32),
        pltpu.VMEM((_BW,), jnp.float32),
        pltpu.VMEM((_BW,), jnp.float32),
        pltpu.SemaphoreType.DMA,
        pltpu.SemaphoreType.DMA,
        pltpu.SemaphoreType.DMA,
        pltpu.SemaphoreType.DMA,
        pltpu.SemaphoreType.DMA,
        pltpu.SemaphoreType.DMA,
    ),
)
def _gather_all(etT, rel2, diag, curv1, ctx2, hb1, tb1, head, tail, rel, neg,
                head_o, tail_o, rel_o, neg_o, curv_o, diag_o, ctx_o,
                hb_o, tb_o, ntb_o,
                ridx, rp, hidx, tidx, nidx, hfull, tfull, nfull,
                hits, hits2, colsm, colstart, coloff, chunk0, chunk1, ring,
                lineA, rrow, xrow, crow, hbrow, tbrow, ntbrow,
                gsem, semA, csem0, csem1, rowsem, ssem):
    wid = lax.axis_index("s") * _NC + lax.axis_index("c")
    base = wid * _BW
    sl = pl.ds(base, _BW)
    iota16 = lax.iota(jnp.int32, 16)

    pltpu.sync_copy(rel.at[sl], ridx)
    pltpu.sync_copy(head.at[sl], hidx)
    pltpu.sync_copy(tail.at[sl], tidx)
    pltpu.sync_copy(neg.at[sl], nidx)

    def mkpairs(i, _):
        s16 = pl.ds(i * 16, 16)
        rp[s16] = ridx[s16] >> 1
        return 0
    lax.fori_loop(0, _BW // 16, mkpairs, 0, unroll=True)

    small = [
        pltpu.async_copy(curv1.at[ridx], crow, gsem),
        pltpu.async_copy(hb1.at[hidx], hbrow, gsem),
        pltpu.async_copy(tb1.at[tidx], tbrow, gsem),
        pltpu.async_copy(tb1.at[nidx], ntbrow, gsem),
    ]
    dA = pltpu.async_copy(rel2.at[rp], lineA, semA)

    def sel_table(line, idxv, row):
        for k in range(_BW // 16):
            hvec = iota16 + (16 * k)
            colb = (idxv[pl.ds(16 * k, 16)] & 1) * HIDDEN

            def cbody(c, _):
                v = plsc.load_gather(line, [hvec, colb + c])
                cvec = jnp.full((16,), c, dtype=jnp.int32)
                plsc.store_scatter(row, [hvec, cvec], v)
                return 0
            lax.fori_loop(0, HIDDEN, cbody, 0)

    pltpu.sync_copy(head, hfull)
    pltpu.sync_copy(tail, tfull)
    pltpu.sync_copy(neg, nfull)

    t0 = (wid * _NTC) // _NW
    t1 = ((wid + 1) * _NTC) // _NW

    def build(tblref, tblid, kcnt0):
        def chunk(j, kcnt):
            r = tblref[pl.ds(j * 16, 16)]
            tc = r >> 7
            m = (tc >= t0) & (tc < t1)
            n = lax.reduce_max(plsc.all_reduce_population_count(m), (0,))

            def have():
                pos = kcnt + plsc.cumsum(m.astype(jnp.int32)) - 1
                ea = ((tc - t0) << 7) | (r & 127)
                eb = (16 * j + iota16) | (tblid << 12)
                plsc.store_scatter(hits, [pos], ea | (eb << 15), mask=m)
            pl.when(n > 0)(have)
            return kcnt + n
        return lax.fori_loop(0, BATCH // 16, chunk, kcnt0)

    kcnt = build(hfull, 0, jnp.int32(0))
    kcnt = build(tfull, 1, kcnt)
    kcnt = build(nfull, 2, kcnt)
    kchunks = (kcnt + 15) >> 4

    ncols = t1 - t0
    dummy = hb1.at[pl.ds(0, HIDDEN)]
    zeros16 = jnp.zeros((16,), jnp.int32)

    def zerocol(i, _):
        colsm[i] = 0
        return 0
    lax.fori_loop(0, 256, zerocol, 0)

    def count_chunk(j, _):
        a = hits[pl.ds(j * 16, 16)]
        valid = ((j * 16 + iota16) < kcnt).astype(jnp.int32)
        trel = (a & 0x7FFF) >> 7
        for i in range(16):
            colsm[trel[i]] = colsm[trel[i]] + valid[i]
        return 0
    lax.fori_loop(0, kchunks, count_chunk, 0)

    def prefix(i, run):
        colstart[i] = run
        coloff[i] = run
        return run + colsm[i]
    lax.fori_loop(0, 256, prefix, jnp.int32(0))

    def place_chunk(j, _):
        a = hits[pl.ds(j * 16, 16)]
        validb = (j * 16 + iota16) < kcnt
        valid = validb.astype(jnp.int32)
        trel = (a & 0x7FFF) >> 7
        pos = zeros16
        for i in range(16):
            p = coloff[trel[i]]
            coloff[trel[i]] = p + valid[i]
            pos = jnp.where(iota16 == i, p, pos)
        plsc.store_scatter(hits2, [pos], a, mask=validb)
        return 0
    lax.fori_loop(0, kchunks, place_chunk, 0)

    dA.wait()
    sel_table(lineA, ridx, rrow)
    dX = pltpu.async_copy(ctx2.at[rp], lineA, semA)
    st_rel = pltpu.async_copy(rrow, rel_o.at[sl], ssem)
    dX.wait()
    sel_table(lineA, ridx, xrow)
    dD = pltpu.async_copy(diag.at[ridx], lineA, semA)
    st_ctx = pltpu.async_copy(xrow, ctx_o.at[sl], ssem)

    def process_col(seg_base, seg_cnt, chunk, loff, state):
        def kchunk(j, st):
            a = hits2[pl.ds(seg_base + j * 16, 16)]
            m = (j * 16 + iota16) < seg_cnt
            n = jnp.minimum(seg_cnt - j * 16, 16)

            def have(st2):
                iss2, drn2 = st2
                cur = iss2
                for i in range(16):
                    mi = (j * 16 + i) < seg_cnt
                    e = a[i]
                    b = (e >> 15) & 4095
                    tbl = e >> 27
                    slot = cur & (_RING - 1)

                    @pl.when(mi)
                    def _():
                        lbc = jnp.full((16,), (e & 127) + loff,
                                       dtype=jnp.int32)
                        for q in range(4):
                            v = plsc.load_gather(chunk, [iota16 + 16 * q, lbc])
                            ring[slot, pl.ds(16 * q, 16)] = v
                    src = ring.at[slot]
                    dst = pl.ds(b * HIDDEN, HIDDEN)

                    @pl.when(mi & (tbl == 0))
                    def _():
                        pltpu.async_copy(src, head_o.at[dst], rowsem)

                    @pl.when(mi & (tbl == 1))
                    def _():
                        pltpu.async_copy(src, tail_o.at[dst], rowsem)

                    @pl.when(mi & (tbl == 2))
                    def _():
                        pltpu.async_copy(src, neg_o.at[dst], rowsem)
                    cur = cur + mi.astype(jnp.int32)
                iss2 = cur

                def drain_some(st4):
                    iss4, drn4 = st4

                    def dr(i2, d):
                        pltpu.make_async_copy(dummy, ring.at[0], rowsem).wait()
                        return d + 1
                    drn4 = lax.fori_loop(0, 64, dr, drn4)
                    return (iss4, drn4)
                return lax.cond(iss2 - drn2 >= _RING - 32, drain_some,
                                lambda s: s, (iss2, drn2))
            return lax.cond(n > 0, have, lambda s: s, st)
        return lax.fori_loop(0, (seg_cnt + 15) >> 4, kchunk, state)

    def ccol(c):
        return jnp.minimum(c, ncols - 1)

    pltpu.async_copy(etT.at[:, pl.ds((t0 + ccol(0)) * 128, 128)],
                     chunk0, csem0)

    def pairbody(i, st):
        cA = ccol(2 * i)
        cB = ccol(2 * i + 1)
        pltpu.async_copy(etT.at[:, pl.ds((t0 + cB) * 128, 128)],
                         chunk1, csem1)
        pltpu.make_async_copy(etT.at[:, pl.ds(0, 128)], chunk0, csem0).wait()
        st = process_col(colstart[cA], colsm[cA], chunk0, 0, st)
        pltpu.async_copy(etT.at[:, pl.ds((t0 + ccol(2 * i + 2)) * 128, 128)],
                         chunk0, csem0)
        pltpu.make_async_copy(etT.at[:, pl.ds(0, 128)], chunk1, csem1).wait()
        return process_col(colstart[cB], colsm[cB], chunk1, 0, st)

    issued, drained = lax.fori_loop(0, (_NTC // _NW) // 2 + 1, pairbody,
                                    (jnp.int32(0), jnp.int32(0)))
    pltpu.make_async_copy(etT.at[:, pl.ds(0, 128)], chunk0, csem0).wait()

    def drfin(i, d):
        pltpu.make_async_copy(dummy, ring.at[0], rowsem).wait()
        return d + 1
    lax.fori_loop(0, issued - drained, drfin, drained)

    for g in small:
        g.wait()
    dD.wait()
    stores = [
        st_rel, st_ctx,
        pltpu.async_copy(crow, curv_o.at[sl], ssem),
        pltpu.async_copy(lineA, diag_o.at[sl], ssem),
        pltpu.async_copy(hbrow, hb_o.at[sl], ssem),
        pltpu.async_copy(tbrow, tb_o.at[sl], ssem),
        pltpu.async_copy(ntbrow, ntb_o.at[sl], ssem),
    ]
    for s in stores:
        s.wait()


def kernel(entity_emb, relation_emb, relation_diag, curvature, context,
           head_bias, tail_bias, head, tail, rel, neg):
    scale = jnp.array([0.125], dtype=jnp.float32)
    (head_e, tail_e, rel_e, neg_e, curv, rel_diag, ctx,
     h_bias, t_bias, neg_t_bias) = _gather_all(
        entity_emb.T,
        relation_emb.reshape(N_RELATION // 2, 2 * HIDDEN),
        relation_diag,
        curvature.reshape(N_RELATION),
        context.reshape(N_RELATION // 2, 2 * HIDDEN),
        head_bias.reshape(N_ENTITY), tail_bias.reshape(N_ENTITY),
        head.astype(jnp.int32), tail.astype(jnp.int32),
        rel.astype(jnp.int32), neg.astype(jnp.int32))
    return (scale, head_e.reshape(BATCH, HIDDEN), tail_e.reshape(BATCH, HIDDEN),
            rel_e, neg_e.reshape(BATCH, HIDDEN),
            curv.reshape(BATCH, 1), rel_diag, ctx,
            h_bias.reshape(BATCH, 1), t_bias.reshape(BATCH, 1),
            neg_t_bias.reshape(BATCH, 1))


# speedup vs baseline: 1.5648x; 1.0298x over previous
import functools

import jax
import jax.numpy as jnp
from jax import lax
from jax.experimental import pallas as pl
from jax.experimental.pallas import tpu as pltpu
from jax.experimental.pallas import tpu_sc as plsc

N_ENTITY = 1000000
N_RELATION = 1000
HIDDEN = 64
BATCH = 4096

_NC, _NS = 2, 16
_NW = _NC * _NS
_BW = BATCH // _NW
_NTC = (N_ENTITY + 127) // 128
_KCAP = 3 * BATCH
_RING = 128

_mesh = plsc.VectorSubcoreMesh(core_axis_name="c", subcore_axis_name="s")


@functools.partial(
    pl.kernel,
    mesh=_mesh,
    compiler_params=pltpu.CompilerParams(use_tc_tiling_on_sc=True,
                                         needs_layout_passes=False),
    out_type=(
        jax.ShapeDtypeStruct((BATCH * HIDDEN,), jnp.float32),
        jax.ShapeDtypeStruct((BATCH * HIDDEN,), jnp.float32),
        jax.ShapeDtypeStruct((BATCH, HIDDEN), jnp.float32),
        jax.ShapeDtypeStruct((BATCH * HIDDEN,), jnp.float32),
        jax.ShapeDtypeStruct((BATCH,), jnp.float32),
        jax.ShapeDtypeStruct((BATCH, 2 * HIDDEN), jnp.float32),
        jax.ShapeDtypeStruct((BATCH, HIDDEN), jnp.float32),
        jax.ShapeDtypeStruct((BATCH,), jnp.float32),
        jax.ShapeDtypeStruct((BATCH,), jnp.float32),
        jax.ShapeDtypeStruct((BATCH,), jnp.float32),
    ),
    scratch_types=(
        pltpu.VMEM((_BW,), jnp.int32),
        pltpu.VMEM((_BW,), jnp.int32),
        pltpu.VMEM((_BW,), jnp.int32),
        pltpu.VMEM((_BW,), jnp.int32),
        pltpu.VMEM((_BW,), jnp.int32),
        pltpu.VMEM((BATCH,), jnp.int32),
        pltpu.VMEM((BATCH,), jnp.int32),
        pltpu.VMEM((BATCH,), jnp.int32),
        pltpu.VMEM((_KCAP,), jnp.int32),
        pltpu.VMEM((_KCAP,), jnp.int32),
        pltpu.SMEM((256,), jnp.int32),
        pltpu.SMEM((256,), jnp.int32),
        pltpu.SMEM((256,), jnp.int32),
        pltpu.VMEM((64, 128), jnp.float32),
        pltpu.VMEM((64, 128), jnp.float32),
        pltpu.VMEM((_RING, HIDDEN), jnp.float32),
        pltpu.VMEM((_BW, 128), jnp.float32),
        pltpu.VMEM((_BW, HIDDEN), jnp.float32),
        pltpu.VMEM((_BW, HIDDEN), jnp.float32),
        pltpu.VMEM((_BW,), jnp.float32),
        pltpu.VMEM((_BW,), jnp.float32),
        pltpu.VMEM((_BW,), jnp.float32),
        pltpu.VMEM((_BW,), jnp.float32),
        pltpu.SemaphoreType.DMA,
        pltpu.SemaphoreType.DMA,
        pltpu.SemaphoreType.DMA,
        pltpu.SemaphoreType.DMA,
        pltpu.SemaphoreType.DMA,
        pltpu.SemaphoreType.DMA,
    ),
)
def _gather_all(etT, rel2, diag, curv1, ctx2, hb1, tb1, head, tail, rel, neg,
                head_o, tail_o, rel_o, neg_o, curv_o, diag_o, ctx_o,
                hb_o, tb_o, ntb_o,
                ridx, rp, hidx, tidx, nidx, hfull, tfull, nfull,
                hits, hits2, colsm, colstart, coloff, chunk0, chunk1, ring,
                lineA, rrow, xrow, crow, hbrow, tbrow, ntbrow,
                gsem, semA, csem0, csem1, rowsem, ssem):
    wid = lax.axis_index("s") * _NC + lax.axis_index("c")
    base = wid * _BW
    sl = pl.ds(base, _BW)
    iota16 = lax.iota(jnp.int32, 16)

    pltpu.sync_copy(rel.at[sl], ridx)
    pltpu.sync_copy(head.at[sl], hidx)
    pltpu.sync_copy(tail.at[sl], tidx)
    pltpu.sync_copy(neg.at[sl], nidx)

    def mkpairs(i, _):
        s16 = pl.ds(i * 16, 16)
        rp[s16] = ridx[s16] >> 1
        return 0
    lax.fori_loop(0, _BW // 16, mkpairs, 0, unroll=True)

    small = [
        pltpu.async_copy(curv1.at[ridx], crow, gsem),
        pltpu.async_copy(hb1.at[hidx], hbrow, gsem),
        pltpu.async_copy(tb1.at[tidx], tbrow, gsem),
        pltpu.async_copy(tb1.at[nidx], ntbrow, gsem),
    ]
    dA = pltpu.async_copy(rel2.at[rp], lineA, semA)

    def sel_table(line, idxv, row):
        for k in range(_BW // 16):
            hvec = iota16 + (16 * k)
            colb = (idxv[pl.ds(16 * k, 16)] & 1) * HIDDEN

            def cbody(c, _):
                v = plsc.load_gather(line, [hvec, colb + c])
                cvec = jnp.full((16,), c, dtype=jnp.int32)
                plsc.store_scatter(row, [hvec, cvec], v)
                return 0
            lax.fori_loop(0, HIDDEN, cbody, 0)

    pltpu.sync_copy(head, hfull)
    pltpu.sync_copy(tail, tfull)
    pltpu.sync_copy(neg, nfull)

    t0 = (wid * _NTC) // _NW
    t1 = ((wid + 1) * _NTC) // _NW

    def build(tblref, tblid, kcnt0):
        def chunk(j, kcnt):
            r = tblref[pl.ds(j * 16, 16)]
            tc = r >> 7
            m = (tc >= t0) & (tc < t1)
            n = lax.reduce_max(plsc.all_reduce_population_count(m), (0,))

            def have():
                pos = kcnt + plsc.cumsum(m.astype(jnp.int32)) - 1
                ea = ((tc - t0) << 7) | (r & 127)
                eb = (16 * j + iota16) | (tblid << 12)
                plsc.store_scatter(hits, [pos], ea | (eb << 15), mask=m)
            pl.when(n > 0)(have)
            return kcnt + n
        return lax.fori_loop(0, BATCH // 16, chunk, kcnt0)

    kcnt = build(hfull, 0, jnp.int32(0))
    kcnt = build(tfull, 1, kcnt)
    kcnt = build(nfull, 2, kcnt)
    kchunks = (kcnt + 15) >> 4

    ncols = t1 - t0
    dummy = hb1.at[pl.ds(0, HIDDEN)]
    zeros16 = jnp.zeros((16,), jnp.int32)

    def zerocol(i, _):
        colsm[i] = 0
        return 0
    lax.fori_loop(0, 256, zerocol, 0)

    def count_chunk(j, _):
        a = hits[pl.ds(j * 16, 16)]
        valid = ((j * 16 + iota16) < kcnt).astype(jnp.int32)
        trel = (a & 0x7FFF) >> 7
        for i in range(16):
            colsm[trel[i]] = colsm[trel[i]] + valid[i]
        return 0
    lax.fori_loop(0, kchunks, count_chunk, 0)

    def prefix(i, run):
        colstart[i] = run
        coloff[i] = run
        return run + colsm[i]
    lax.fori_loop(0, 256, prefix, jnp.int32(0))

    def place_chunk(j, _):
        a = hits[pl.ds(j * 16, 16)]
        validb = (j * 16 + iota16) < kcnt
        valid = validb.astype(jnp.int32)
        trel = (a & 0x7FFF) >> 7
        pos = zeros16
        for i in range(16):
            p = coloff[trel[i]]
            coloff[trel[i]] = p + valid[i]
            pos = jnp.where(iota16 == i, p, pos)
        plsc.store_scatter(hits2, [pos], a, mask=validb)
        return 0
    lax.fori_loop(0, kchunks, place_chunk, 0)

    dA.wait()
    sel_table(lineA, ridx, rrow)
    dX = pltpu.async_copy(ctx2.at[rp], lineA, semA)
    st_rel = pltpu.async_copy(rrow, rel_o.at[sl], ssem)
    dX.wait()
    sel_table(lineA, ridx, xrow)
    dD = pltpu.async_copy(diag.at[ridx], lineA, semA)
    st_ctx = pltpu.async_copy(xrow, ctx_o.at[sl], ssem)

    def process_col(seg_base, seg_cnt, chunk, loff, state):
        def kchunk(j, st):
            a = hits2[pl.ds(seg_base + j * 16, 16)]
            m = (j * 16 + iota16) < seg_cnt
            n = jnp.minimum(seg_cnt - j * 16, 16)

            def have(st2):
                iss2, drn2 = st2
                cur = iss2
                for i in range(16):
                    mi = (j * 16 + i) < seg_cnt
                    e = a[i]
                    b = (e >> 15) & 4095
                    tbl = e >> 27
                    slot = cur & (_RING - 1)

                    @pl.when(mi)
                    def _():
                        lbc = jnp.full((16,), (e & 127) + loff,
                                       dtype=jnp.int32)
                        for q in range(4):
                            v = plsc.load_gather(chunk, [iota16 + 16 * q, lbc])
                            ring[slot, pl.ds(16 * q, 16)] = v
                    src = ring.at[slot]
                    dst = pl.ds(b * HIDDEN, HIDDEN)

                    @pl.when(mi & (tbl == 0))
                    def _():
                        pltpu.async_copy(src, head_o.at[dst], rowsem)

                    @pl.when(mi & (tbl == 1))
                    def _():
                        pltpu.async_copy(src, tail_o.at[dst], rowsem)

                    @pl.when(mi & (tbl == 2))
                    def _():
                        pltpu.async_copy(src, neg_o.at[dst], rowsem)
                    cur = cur + mi.astype(jnp.int32)
                iss2 = cur

                def drain_some(st4):
                    iss4, drn4 = st4

                    def dr(i2, d):
                        pltpu.make_async_copy(dummy, ring.at[0], rowsem).wait()
                        return d + 1
                    drn4 = lax.fori_loop(0, 64, dr, drn4)
                    return (iss4, drn4)
                return lax.cond(iss2 - drn2 >= _RING - 32, drain_some,
                                lambda s: s, (iss2, drn2))
            return lax.cond(n > 0, have, lambda s: s, st)
        return lax.fori_loop(0, (seg_cnt + 15) >> 4, kchunk, state)

    def want(c):
        return (c < ncols) & (colsm[jnp.minimum(c, 255)] > 0)

    @pl.when(want(0))
    def _():
        pltpu.async_copy(etT.at[:, pl.ds(t0 * 128, 128)], chunk0, csem0)

    def pairbody(i, st):
        cA = 2 * i
        cB = cA + 1

        @pl.when(want(cB))
        def _():
            pltpu.async_copy(etT.at[:, pl.ds((t0 + cB) * 128, 128)],
                             chunk1, csem1)

        def doA(st2):
            pltpu.make_async_copy(etT.at[:, pl.ds(0, 128)], chunk0,
                                  csem0).wait()
            return process_col(colstart[cA], colsm[cA], chunk0, 0, st2)
        st = lax.cond(want(cA), doA, lambda s2: s2, st)

        @pl.when(want(cB + 1))
        def _():
            pltpu.async_copy(etT.at[:, pl.ds((t0 + cB + 1) * 128, 128)],
                             chunk0, csem0)

        def doB(st2):
            pltpu.make_async_copy(etT.at[:, pl.ds(0, 128)], chunk1,
                                  csem1).wait()
            return process_col(colstart[cB], colsm[cB], chunk1, 0, st2)
        return lax.cond(want(cB), doB, lambda s2: s2, st)

    issued, drained = lax.fori_loop(0, (_NTC // _NW) // 2 + 2, pairbody,
                                    (jnp.int32(0), jnp.int32(0)))

    def drfin(i, d):
        pltpu.make_async_copy(dummy, ring.at[0], rowsem).wait()
        return d + 1
    lax.fori_loop(0, issued - drained, drfin, drained)

    for g in small:
        g.wait()
    dD.wait()
    stores = [
        st_rel, st_ctx,
        pltpu.async_copy(crow, curv_o.at[sl], ssem),
        pltpu.async_copy(lineA, diag_o.at[sl], ssem),
        pltpu.async_copy(hbrow, hb_o.at[sl], ssem),
        pltpu.async_copy(tbrow, tb_o.at[sl], ssem),
        pltpu.async_copy(ntbrow, ntb_o.at[sl], ssem),
    ]
    for s in stores:
        s.wait()


def kernel(entity_emb, relation_emb, relation_diag, curvature, context,
           head_bias, tail_bias, head, tail, rel, neg):
    scale = jnp.array([0.125], dtype=jnp.float32)
    (head_e, tail_e, rel_e, neg_e, curv, rel_diag, ctx,
     h_bias, t_bias, neg_t_bias) = _gather_all(
        entity_emb.T,
        relation_emb.reshape(N_RELATION // 2, 2 * HIDDEN),
        relation_diag,
        curvature.reshape(N_RELATION),
        context.reshape(N_RELATION // 2, 2 * HIDDEN),
        head_bias.reshape(N_ENTITY), tail_bias.reshape(N_ENTITY),
        head.astype(jnp.int32), tail.astype(jnp.int32),
        rel.astype(jnp.int32), neg.astype(jnp.int32))
    return (scale, head_e.reshape(BATCH, HIDDEN), tail_e.reshape(BATCH, HIDDEN),
            rel_e, neg_e.reshape(BATCH, HIDDEN),
            curv.reshape(BATCH, 1), rel_diag, ctx,
            h_bias.reshape(BATCH, 1), t_bias.reshape(BATCH, 1),
            neg_t_bias.reshape(BATCH, 1))
